# baseline (speedup 1.0000x reference)
import jax
import jax.numpy as jnp
from jax import lax
from jax.experimental import pallas as pl
from jax.experimental.pallas import tpu as pltpu
from jax.experimental.pallas import tpu_sc as plsc

_HU = (-1000.0, -900.0, -400.0, -100.0, -50.0, -10.0,
       20.0, 40.0, 60.0, 100.0, 800.0, 1000.0)

_NROW = 512
_NGROUP = 256
_QROW = 128


def _sc_body(x_hbm, tabp_hbm, out_hbm, xq, binsC, pairIdx, gbuf,
             obufA, obufB, stab, semA, semB, semG):
    cid = lax.axis_index("c")
    sid = lax.axis_index("s")
    wid = sid * 2 + cid

    lanes = lax.iota(jnp.int32, 16)
    zero16 = lanes * 0
    p0 = (lanes // 8) * 16 + (lanes % 8) * 2

    @pl.when(sid == 0)
    def _():
        pltpu.sync_copy(tabp_hbm, stab)

    plsc.subcore_barrier()

    for qt in range(4):
        pltpu.sync_copy(x_hbm.at[pl.ds(qt * _QROW, _QROW), wid, :], xq)

        def blk(gsub, carry, qt=qt):

            @plsc.parallel_loop(0, _QROW, unroll=4)
            def p1(i):
                xr = xq[i, pl.ds(gsub * 16, 16)]
                b = zero16 + 1
                for hu in _HU[1:]:
                    b = b + (xr >= hu).astype(jnp.int32)
                binsC[pl.ds(i * 16, 16)] = b

            @plsc.parallel_loop(0, 64, unroll=4)
            def pbuild(k):
                idxv = p0 + 32 * k
                b0 = plsc.load_gather(binsC, [idxv])
                b1 = plsc.load_gather(binsC, [idxv + 1])
                pairIdx[pl.ds(k * 16, 16)] = b0 * 13 + b1

            grp = wid * 8 + gsub
            dst = out_hbm.at[pl.ds(qt * 16, 16), grp]

            for par, obuf, sem in ((0, obufA, semA), (1, obufB, semB)):

                @pl.when(lax.bitwise_and(gsub, 1) == par)
                def _(obuf=obuf, sem=sem):
                    cp = pltpu.make_async_copy(obuf, dst, sem)
                    if qt == 0:
                        @pl.when(gsub >= 2)
                        def _():
                            cp.wait()
                    else:
                        cp.wait()

                    for m in range(8):
                        gcp = pltpu.async_copy(
                            stab.at[pairIdx.at[pl.ds(m * 128, 128)]],
                            gbuf, semG)
                        gcp.wait()

                        @plsc.parallel_loop(0, 128, unroll=4)
                        def relay(jj, m=m):
                            u = m * 128 + jj
                            t = gbuf[jj, :]
                            obuf[u // 64, lax.bitwise_and(u // 8, 7),
                                 pl.ds(lax.bitwise_and(u, 7) * 16, 16)] = t

                    cp.start()
            return carry

        lax.fori_loop(0, 8, blk, 0)

    for obuf, sem in ((obufA, semA), (obufB, semB)):
        pltpu.make_async_copy(obuf, out_hbm.at[pl.ds(0, 16), 0], sem).wait()


@jax.jit
def _run(x3, tabp):
    mesh = plsc.VectorSubcoreMesh(core_axis_name="c", subcore_axis_name="s",
                                  num_cores=2, num_subcores=16)
    return pl.kernel(
        _sc_body,
        out_type=jax.ShapeDtypeStruct((64, _NGROUP, 8, 128), jnp.float32),
        mesh=mesh,
        compiler_params=pltpu.CompilerParams(needs_layout_passes=False),
        scratch_types=[
            pltpu.VMEM((_QROW, 128), jnp.float32),
            pltpu.VMEM((16 * _QROW,), jnp.int32),
            pltpu.VMEM((1024,), jnp.int32),
            pltpu.VMEM((128, 16), jnp.float32),
            pltpu.VMEM((16, 8, 128), jnp.float32),
            pltpu.VMEM((16, 8, 128), jnp.float32),
            pltpu.VMEM_SHARED((169, 16), jnp.float32),
            pltpu.SemaphoreType.DMA,
            pltpu.SemaphoreType.DMA,
            pltpu.SemaphoreType.DMA,
        ],
    )(x3, tabp)


def kernel(x, vectors):
    x3 = x.reshape(_NROW, 32, 128)
    v = vectors.astype(jnp.float32)
    tabp = jnp.concatenate(
        [jnp.repeat(v, 13, axis=0), jnp.tile(v, (13, 1))], axis=1)
    out4 = _run(x3, tabp)
    out6 = out4.reshape(1, 8, 8, _NGROUP, 8, 128)
    outT = jnp.transpose(out6, (0, 3, 5, 1, 2, 4))
    return outT.reshape(1, 32768, 8, 8, 8)


# speedup vs baseline: 1.1606x; 1.1606x over previous
import jax
import jax.numpy as jnp
from jax import lax
from jax.experimental import pallas as pl
from jax.experimental.pallas import tpu as pltpu
from jax.experimental.pallas import tpu_sc as plsc

_HU = (-1000.0, -900.0, -400.0, -100.0, -50.0, -10.0,
       20.0, 40.0, 60.0, 100.0, 800.0, 1000.0)

_NROW = 512
_NGROUP = 256
_QROW = 128


def _sc_body(x_hbm, tabp_hbm, out_hbm, xq, binsC, pairIdx, gbuf,
             obufA, obufB, stab, semA, semB, semG, semG2):
    cid = lax.axis_index("c")
    sid = lax.axis_index("s")
    wid = sid * 2 + cid

    lanes = lax.iota(jnp.int32, 16)
    zero16 = lanes * 0
    p0 = (lanes // 8) * 16 + (lanes % 8) * 2

    @pl.when(sid == 0)
    def _():
        pltpu.sync_copy(tabp_hbm, stab)

    plsc.subcore_barrier()

    for qt in range(4):
        pltpu.sync_copy(x_hbm.at[pl.ds(qt * _QROW, _QROW), wid, :], xq)

        def blk(gsub, carry, qt=qt):

            @plsc.parallel_loop(0, _QROW, unroll=4)
            def p1(i):
                xr = xq[i, pl.ds(gsub * 16, 16)]
                b = zero16 + 1
                for hu in _HU[1:]:
                    b = b + (xr >= hu).astype(jnp.int32)
                binsC[pl.ds(i * 16, 16)] = b

            @plsc.parallel_loop(0, 64, unroll=4)
            def pbuild(k):
                idxv = p0 + 32 * k
                b0 = plsc.load_gather(binsC, [idxv])
                b1 = plsc.load_gather(binsC, [idxv + 1])
                pairIdx[pl.ds(k * 16, 16)] = b0 * 13 + b1

            grp = wid * 8 + gsub
            dst = out_hbm.at[pl.ds(qt * 16, 16), grp]

            for par, obuf, sem in ((0, obufA, semA), (1, obufB, semB)):

                @pl.when(lax.bitwise_and(gsub, 1) == par)
                def _(obuf=obuf, sem=sem):
                    cp = pltpu.make_async_copy(obuf, dst, sem)
                    pltpu.async_copy(
                        stab.at[pairIdx.at[pl.ds(0, 64)]],
                        gbuf.at[pl.ds(0, 64)], semG)
                    if qt == 0:
                        @pl.when(gsub >= 2)
                        def _():
                            cp.wait()
                    else:
                        cp.wait()

                    def chunk(m, carry2, obuf=obuf):
                        for par2, gsem in ((0, semG), (1, semG2)):

                            @pl.when(lax.bitwise_and(m, 1) == par2)
                            def _(par2=par2, gsem=gsem):
                                osem = (semG2, semG)[par2]
                                pltpu.make_async_copy(
                                    stab.at[pairIdx.at[pl.ds(0, 64)]],
                                    gbuf.at[pl.ds(par2 * 64, 64)],
                                    gsem).wait()

                                @pl.when(m < 15)
                                def _():
                                    pltpu.async_copy(
                                        stab.at[pairIdx.at[
                                            pl.ds((m + 1) * 64, 64)]],
                                        gbuf.at[pl.ds((1 - par2) * 64, 64)],
                                        osem)

                                @plsc.parallel_loop(0, 64, unroll=4)
                                def relay(jj):
                                    u = m * 64 + jj
                                    t = gbuf[par2 * 64 + jj, :]
                                    obuf[u // 64,
                                         lax.bitwise_and(u // 8, 7),
                                         pl.ds(lax.bitwise_and(u, 7) * 16,
                                               16)] = t
                        return carry2

                    lax.fori_loop(0, 16, chunk, 0)

                    cp.start()
            return carry

        lax.fori_loop(0, 8, blk, 0)

    for obuf, sem in ((obufA, semA), (obufB, semB)):
        pltpu.make_async_copy(obuf, out_hbm.at[pl.ds(0, 16), 0], sem).wait()


@jax.jit
def _run(x3, tabp):
    mesh = plsc.VectorSubcoreMesh(core_axis_name="c", subcore_axis_name="s",
                                  num_cores=2, num_subcores=16)
    return pl.kernel(
        _sc_body,
        out_type=jax.ShapeDtypeStruct((64, _NGROUP, 8, 128), jnp.float32),
        mesh=mesh,
        compiler_params=pltpu.CompilerParams(needs_layout_passes=False),
        scratch_types=[
            pltpu.VMEM((_QROW, 128), jnp.float32),
            pltpu.VMEM((16 * _QROW,), jnp.int32),
            pltpu.VMEM((1024,), jnp.int32),
            pltpu.VMEM((128, 16), jnp.float32),
            pltpu.VMEM((16, 8, 128), jnp.float32),
            pltpu.VMEM((16, 8, 128), jnp.float32),
            pltpu.VMEM_SHARED((169, 16), jnp.float32),
            pltpu.SemaphoreType.DMA,
            pltpu.SemaphoreType.DMA,
            pltpu.SemaphoreType.DMA,
            pltpu.SemaphoreType.DMA,
        ],
    )(x3, tabp)


def kernel(x, vectors):
    x3 = x.reshape(_NROW, 32, 128)
    v = vectors.astype(jnp.float32)
    tabp = jnp.concatenate(
        [jnp.repeat(v, 13, axis=0), jnp.tile(v, (13, 1))], axis=1)
    out4 = _run(x3, tabp)
    out6 = out4.reshape(1, 8, 8, _NGROUP, 8, 128)
    outT = jnp.transpose(out6, (0, 3, 5, 1, 2, 4))
    return outT.reshape(1, 32768, 8, 8, 8)
